# Initial kernel scaffold; baseline (speedup 1.0000x reference)
import jax
import jax.numpy as jnp
from jax.experimental import pallas as pl


def kernel(x, centroids):
    raise NotImplementedError("write your pallas kernel here")


# speedup vs baseline: 3.2138x; 3.2138x over previous
import functools

import jax
import jax.numpy as jnp
from jax.experimental import pallas as pl

TOPK = 32


def _fused_kernel(x_ref, c_ref, sims_ref, idx_ref, am_ref, *, inv_sqrt_d, kdim):
    xb = x_ref[0]
    c = c_ref[...]
    logits_t = jax.lax.dot_general(
        c.astype(jnp.bfloat16), xb.astype(jnp.bfloat16),
        (((1,), (1,)), ((), ())),
        preferred_element_type=jnp.float32,
    ) * inv_sqrt_d
    m = jnp.max(logits_t, axis=0, keepdims=True)
    e = jnp.exp(logits_t - m)
    probs_t = e / jnp.sum(e, axis=0, keepdims=True)
    am_ref[...] = probs_t.T

    iota = jax.lax.broadcasted_iota(jnp.int32, probs_t.shape, 0)
    vals = probs_t
    sims_rows = []
    idx_rows = []
    for _ in range(TOPK):
        mx = jnp.max(vals, axis=0, keepdims=True)
        cand = jnp.where(vals == mx, iota, kdim)
        amin = jnp.min(cand, axis=0, keepdims=True)
        sims_rows.append(mx)
        idx_rows.append(amin)
        vals = jnp.where(cand == amin, -1.0, vals)
    sims_ref[0] = jnp.concatenate(sims_rows, axis=0).T
    idx_ref[0] = jnp.concatenate(idx_rows, axis=0).T


@jax.jit
def kernel(x, centroids):
    B, S, D = x.shape
    K = centroids.shape[0]
    S_blk = 256
    grid = (B, S // S_blk)
    body = functools.partial(_fused_kernel,
                             inv_sqrt_d=1.0 / (D ** 0.5), kdim=K)
    sims, indices, amatrix_r = pl.pallas_call(
        body,
        grid=grid,
        in_specs=[
            pl.BlockSpec((1, S_blk, D), lambda b, s: (b, s, 0)),
            pl.BlockSpec((K, D), lambda b, s: (0, 0)),
        ],
        out_specs=[
            pl.BlockSpec((1, S_blk, TOPK), lambda b, s: (b, s, 0)),
            pl.BlockSpec((1, S_blk, TOPK), lambda b, s: (b, s, 0)),
            pl.BlockSpec((S_blk, K), lambda b, s: (s, b)),
        ],
        out_shape=[
            jax.ShapeDtypeStruct((B, S, TOPK), jnp.float32),
            jax.ShapeDtypeStruct((B, S, TOPK), jnp.int32),
            jax.ShapeDtypeStruct((S, B * K), jnp.float32),
        ],
    )(x, centroids)
    return sims, indices, amatrix_r


# speedup vs baseline: 3.5286x; 1.0980x over previous
import functools

import jax
import jax.numpy as jnp
from jax.experimental import pallas as pl

TOPK = 32


def _fused_kernel(x_ref, c_ref, sims_ref, idx_ref, am_ref, *, inv_sqrt_d, kdim):
    xb = x_ref[0]
    c = c_ref[...]
    logits_t = jax.lax.dot_general(
        c.astype(jnp.bfloat16), xb.astype(jnp.bfloat16),
        (((1,), (1,)), ((), ())),
        preferred_element_type=jnp.float32,
    ) * inv_sqrt_d
    m = jnp.max(logits_t, axis=0, keepdims=True)
    e = jnp.exp(logits_t - m)
    probs_t = e / jnp.sum(e, axis=0, keepdims=True)
    am_ref[...] = probs_t.T

    iota = jax.lax.broadcasted_iota(jnp.int32, probs_t.shape, 0)
    vals = probs_t
    sims_rows = []
    idx_rows = []
    for _ in range(TOPK):
        v, ix = vals, iota
        while v.shape[0] > 1:
            h = v.shape[0] // 2
            a_v, b_v = v[:h], v[h:]
            take = a_v >= b_v
            v = jnp.maximum(a_v, b_v)
            ix = jnp.where(take, ix[:h], ix[h:])
        sims_rows.append(v)
        idx_rows.append(ix)
        vals = jnp.where(iota == ix, -1.0, vals)
    sims_ref[0] = jnp.concatenate(sims_rows, axis=0).T
    idx_ref[0] = jnp.concatenate(idx_rows, axis=0).T


@jax.jit
def kernel(x, centroids):
    B, S, D = x.shape
    K = centroids.shape[0]
    S_blk = 512
    grid = (B, S // S_blk)
    body = functools.partial(_fused_kernel,
                             inv_sqrt_d=1.0 / (D ** 0.5), kdim=K)
    sims, indices, amatrix_r = pl.pallas_call(
        body,
        grid=grid,
        in_specs=[
            pl.BlockSpec((1, S_blk, D), lambda b, s: (b, s, 0)),
            pl.BlockSpec((K, D), lambda b, s: (0, 0)),
        ],
        out_specs=[
            pl.BlockSpec((1, S_blk, TOPK), lambda b, s: (b, s, 0)),
            pl.BlockSpec((1, S_blk, TOPK), lambda b, s: (b, s, 0)),
            pl.BlockSpec((S_blk, K), lambda b, s: (s, b)),
        ],
        out_shape=[
            jax.ShapeDtypeStruct((B, S, TOPK), jnp.float32),
            jax.ShapeDtypeStruct((B, S, TOPK), jnp.int32),
            jax.ShapeDtypeStruct((S, B * K), jnp.float32),
        ],
    )(x, centroids)
    return sims, indices, amatrix_r
